# Initial kernel scaffold; baseline (speedup 1.0000x reference)
import jax
import jax.numpy as jnp
from jax.experimental import pallas as pl


def kernel(centers, contexts, neg_samples, emb):
    raise NotImplementedError("write your pallas kernel here")


# speedup vs baseline: 1.8735x; 1.8735x over previous
import functools

import jax
import jax.numpy as jnp
from jax import lax
from jax.experimental import pallas as pl
from jax.experimental.pallas import tpu as pltpu
from jax.experimental.pallas import tpu_sc as plsc

B = 16384
D = 64
K = 3
NC = 2
NS = 16
NW = NC * NS
PER_W = B // NW
CHUNK = 128
NCHUNK = PER_W // CHUNK


def _sc_scores_body(emb, cen, ctx, neg, out,
                    cidx, xidx, nidx, urows, vrows, nrows,
                    pbuf, nb0, nb1, nb2, sem):
    wid = lax.axis_index("s") * NC + lax.axis_index("c")
    lanes = lax.iota(jnp.int32, 16)
    for c in range(NCHUNK):
        base = wid * PER_W + c * CHUNK
        pltpu.sync_copy(cen.at[pl.ds(base, CHUNK)], cidx)
        pltpu.sync_copy(ctx.at[pl.ds(base, CHUNK)], xidx)
        pltpu.sync_copy(neg.at[pl.ds(base * K, CHUNK * K)], nidx)
        d1 = pltpu.async_copy(emb.at[cidx], urows, sem)
        d2 = pltpu.async_copy(emb.at[xidx], vrows, sem)
        d3 = pltpu.async_copy(emb.at[nidx.at[pl.ds(0, 128)]],
                              nrows.at[pl.ds(0, 128)], sem)
        d4 = pltpu.async_copy(emb.at[nidx.at[pl.ds(128, 128)]],
                              nrows.at[pl.ds(128, 128)], sem)
        d5 = pltpu.async_copy(emb.at[nidx.at[pl.ds(256, 128)]],
                              nrows.at[pl.ds(256, 128)], sem)
        d1.wait(); d2.wait(); d3.wait(); d4.wait(); d5.wait()

        for g in range(CHUNK // 16):
            zero = jnp.zeros((16,), jnp.float32)

            def gbody(jj, accs, g=g):
                pa, a0, a1, a2 = accs
                j = g * 16 + jj
                hit = lanes == jj
                u = [urows[j, pl.ds(q * 16, 16)] for q in range(D // 16)]
                v = [vrows[j, pl.ds(q * 16, 16)] for q in range(D // 16)]
                pos = u[0] * v[0] + u[1] * v[1] + u[2] * v[2] + u[3] * v[3]
                pa = jnp.where(hit, jnp.sum(pos), pa)
                j3 = j * K
                neg_accs = []
                for k, ak in enumerate((a0, a1, a2)):
                    m = [nrows[j3 + k, pl.ds(q * 16, 16)]
                         for q in range(D // 16)]
                    s = u[0] * m[0] + u[1] * m[1] + u[2] * m[2] + u[3] * m[3]
                    neg_accs.append(jnp.where(hit, jnp.sum(s), ak))
                return (pa, *neg_accs)

            pa, a0, a1, a2 = lax.fori_loop(0, 16, gbody, (zero,) * 4)
            pbuf[pl.ds(g * 16, 16)] = pa
            nb0[pl.ds(g * 16, 16)] = a0
            nb1[pl.ds(g * 16, 16)] = a1
            nb2[pl.ds(g * 16, 16)] = a2
        pltpu.sync_copy(pbuf, out.at[0, pl.ds(base, CHUNK)])
        pltpu.sync_copy(nb0, out.at[1, pl.ds(base, CHUNK)])
        pltpu.sync_copy(nb1, out.at[2, pl.ds(base, CHUNK)])
        pltpu.sync_copy(nb2, out.at[3, pl.ds(base, CHUNK)])


_sc_scores = functools.partial(
    pl.kernel,
    out_type=jax.ShapeDtypeStruct((K + 1, B), jnp.float32),
    mesh=plsc.VectorSubcoreMesh(
        core_axis_name="c", subcore_axis_name="s",
        num_cores=NC, num_subcores=NS),
    compiler_params=pltpu.CompilerParams(
        needs_layout_passes=False, use_tc_tiling_on_sc=False),
    scratch_types=[
        pltpu.VMEM((CHUNK,), jnp.int32),
        pltpu.VMEM((CHUNK,), jnp.int32),
        pltpu.VMEM((CHUNK * K,), jnp.int32),
        pltpu.VMEM((CHUNK, D), jnp.float32),
        pltpu.VMEM((CHUNK, D), jnp.float32),
        pltpu.VMEM((CHUNK * K, D), jnp.float32),
        pltpu.VMEM((CHUNK,), jnp.float32),
        pltpu.VMEM((CHUNK,), jnp.float32),
        pltpu.VMEM((CHUNK,), jnp.float32),
        pltpu.VMEM((CHUNK,), jnp.float32),
        pltpu.SemaphoreType.DMA,
    ],
)(_sc_scores_body)


def _loss_body(s_ref, o_ref):
    x = s_ref[...]
    row = lax.broadcasted_iota(jnp.int32, x.shape, 0)
    ispos = row == 0
    s = jnp.where(ispos, x, -x)
    ls = jnp.minimum(s, 0.0) - jnp.log1p(jnp.exp(-jnp.abs(s)))
    pos_sum = jnp.sum(jnp.where(ispos, ls, 0.0))
    neg_sum = jnp.sum(jnp.where(ispos, 0.0, ls))
    o_ref[0, 0] = -(pos_sum / B) - (neg_sum / (K * B))


_loss = pl.pallas_call(
    _loss_body,
    out_shape=jax.ShapeDtypeStruct((1, 1), jnp.float32),
    out_specs=pl.BlockSpec(memory_space=pltpu.SMEM),
)


@jax.jit
def _impl(centers, contexts, neg_samples, emb):
    cen = centers.astype(jnp.int32)
    ctx = contexts.astype(jnp.int32)
    neg = neg_samples.astype(jnp.int32).reshape(-1)
    scores = _sc_scores(emb, cen, ctx, neg)
    return _loss(scores)[0, 0]


def kernel(centers, contexts, neg_samples, emb):
    return _impl(centers, contexts, neg_samples, emb)
